# Initial kernel scaffold; baseline (speedup 1.0000x reference)
import jax
import jax.numpy as jnp
from jax.experimental import pallas as pl


def kernel(x, edge_index, MN_edge_index, W1, b1, W2, b2, Wp1, bp1, Wp2, bp2):
    raise NotImplementedError("write your pallas kernel here")


# speedup vs baseline: 8.5930x; 8.5930x over previous
import functools

import jax
import jax.numpy as jnp
from jax import lax
from jax.experimental import pallas as pl
from jax.experimental.pallas import tpu as pltpu
from jax.experimental.pallas import tpu_sc as plsc

N_NODES = 10000
D = 128
H = 128
E = 320000
E_MN = 100000

NPAD = 10240
NW = 32
CHUNK = 128
NCH = (E // NW) // CHUNK + 1
EW = NCH * CHUNK
EPAD = NW * EW
SCHUNK = 64
SNCH = EW // SCHUNK
MNCH = (E_MN // NW) // CHUNK + 1
MNW = MNCH * CHUNK
MNPAD = NW * MNW

_mesh = plsc.VectorSubcoreMesh(core_axis_name="c", subcore_axis_name="s",
                               num_cores=2, num_subcores=16)
ROWS_PER_TILE = NPAD // 16


DEGW = 16


def _deg_body(dst_hbm, out_hbm, dst_v, ones_v, zero_v, accum_sh, isem):
    c = lax.axis_index("c")
    s = lax.axis_index("s")
    icp = pltpu.async_copy(dst_hbm.at[c * 16 + s], dst_v, isem)

    ones16 = jnp.ones((16,), jnp.float32)
    zeros16 = jnp.zeros((16,), jnp.float32)

    def fill(i, _):
        ones_v[i, pl.ds(0, 16)] = ones16
        zero_v[i % zero_v.shape[0], pl.ds(0, 16)] = zeros16
        return ()

    lax.fori_loop(0, CHUNK, fill, (), unroll=8)

    def zcopy(i, _):
        pltpu.sync_copy(
            zero_v, accum_sh.at[pl.ds(s * ROWS_PER_TILE + i * zero_v.shape[0],
                                      zero_v.shape[0])])
        return ()

    lax.fori_loop(0, ROWS_PER_TILE // zero_v.shape[0], zcopy, ())
    icp.wait()
    plsc.subcore_barrier()

    def body(g, _):
        pltpu.sync_copy(ones_v, accum_sh.at[dst_v.at[g]], add=True)
        return ()

    lax.fori_loop(0, NCH, body, ())
    plsc.subcore_barrier()
    pltpu.sync_copy(accum_sh.at[pl.ds(s * ROWS_PER_TILE, ROWS_PER_TILE)],
                    out_hbm.at[c, pl.ds(s * ROWS_PER_TILE, ROWS_PER_TILE)])


_deg_call = functools.partial(
    pl.kernel,
    out_type=jax.ShapeDtypeStruct((2, NPAD, DEGW), jnp.float32),
    mesh=_mesh,
    scratch_types=[
        pltpu.VMEM((NCH, CHUNK), jnp.int32),
        pltpu.VMEM((CHUNK, DEGW), jnp.float32),
        pltpu.VMEM((64, DEGW), jnp.float32),
        pltpu.VMEM_SHARED((NPAD, DEGW), jnp.float32),
        pltpu.SemaphoreType.DMA,
    ],
)(_deg_body)


def _segsum_body(table_hbm, src_hbm, dst_hbm, out_hbm,
                 src_v, dst_v, rows_v, zero_v, accum_sh, gsem, isem):
    c = lax.axis_index("c")
    s = lax.axis_index("s")
    wid = c * 16 + s

    icp = pltpu.async_copy(src_hbm.at[pl.ds(wid * EW, EW)], src_v, isem)
    dcp = pltpu.async_copy(dst_hbm.at[wid], dst_v, isem)

    zeros16 = jnp.zeros((16,), jnp.float32)

    def zbody(i, _):
        zero_v[i // 8, pl.ds((i % 8) * 16, 16)] = zeros16
        return ()

    lax.fori_loop(0, zero_v.shape[0] * 8, zbody, (), unroll=8)

    def zcopy(i, _):
        pltpu.sync_copy(
            zero_v, accum_sh.at[pl.ds(s * ROWS_PER_TILE + i * zero_v.shape[0],
                                      zero_v.shape[0])])
        return ()

    lax.fori_loop(0, ROWS_PER_TILE // zero_v.shape[0], zcopy, ())
    icp.wait()
    dcp.wait()
    plsc.subcore_barrier()

    pltpu.async_copy(table_hbm.at[src_v.at[pl.ds(0, SCHUNK)]], rows_v.at[0],
                     gsem)

    def body(g, _):
        buf = lax.rem(g, 2)
        pltpu.make_async_copy(table_hbm.at[src_v.at[pl.ds(0, SCHUNK)]],
                              rows_v.at[buf], gsem).wait()

        @pl.when(g + 1 < SNCH)
        def _():
            pltpu.async_copy(
                table_hbm.at[src_v.at[pl.ds((g + 1) * SCHUNK, SCHUNK)]],
                rows_v.at[lax.rem(g + 1, 2)], gsem)

        pltpu.sync_copy(rows_v.at[buf], accum_sh.at[dst_v.at[g]], add=True)
        return ()

    lax.fori_loop(0, SNCH, body, ())
    plsc.subcore_barrier()
    pltpu.sync_copy(accum_sh.at[pl.ds(s * ROWS_PER_TILE, ROWS_PER_TILE)],
                    out_hbm.at[c, pl.ds(s * ROWS_PER_TILE, ROWS_PER_TILE)])


_segsum_call = functools.partial(
    pl.kernel,
    out_type=jax.ShapeDtypeStruct((2, NPAD, H), jnp.float32),
    mesh=_mesh,
    scratch_types=[
        pltpu.VMEM((EW,), jnp.int32),
        pltpu.VMEM((SNCH, SCHUNK), jnp.int32),
        pltpu.VMEM((2, SCHUNK, H), jnp.float32),
        pltpu.VMEM((16, H), jnp.float32),
        pltpu.VMEM_SHARED((NPAD, H), jnp.float32),
        pltpu.SemaphoreType.DMA,
        pltpu.SemaphoreType.DMA,
    ],
)(_segsum_body)


def _mn_body(table_hbm, a_hbm, b_hbm, out_hbm,
             a_v, b_v, rows_a, rows_b, gsem, isem):
    c = lax.axis_index("c")
    s = lax.axis_index("s")
    wid = c * 16 + s
    pltpu.async_copy(a_hbm.at[pl.ds(wid * MNW, MNW)], a_v, isem)
    pltpu.async_copy(b_hbm.at[pl.ds(wid * MNW, MNW)], b_v, isem)
    pltpu.make_async_copy(a_hbm.at[pl.ds(wid * MNW, MNW)], a_v, isem).wait()
    pltpu.make_async_copy(b_hbm.at[pl.ds(wid * MNW, MNW)], b_v, isem).wait()

    def body(g, _):
        pltpu.async_copy(table_hbm.at[a_v.at[pl.ds(g * CHUNK, CHUNK)]],
                         rows_a, gsem)
        pltpu.async_copy(table_hbm.at[b_v.at[pl.ds(g * CHUNK, CHUNK)]],
                         rows_b, gsem)
        pltpu.make_async_copy(table_hbm.at[a_v.at[pl.ds(0, CHUNK)]],
                              rows_a, gsem).wait()
        pltpu.make_async_copy(table_hbm.at[b_v.at[pl.ds(0, CHUNK)]],
                              rows_b, gsem).wait()

        def add_body(r, _):
            for k in range(H // 16):
                sl = pl.ds(k * 16, 16)
                rows_a[r, sl] = rows_a[r, sl] + rows_b[r, sl]
            return ()

        lax.fori_loop(0, CHUNK, add_body, ())
        pltpu.sync_copy(rows_a,
                        out_hbm.at[pl.ds(wid * MNW + g * CHUNK, CHUNK)])
        return ()

    lax.fori_loop(0, MNCH, body, ())


_mn_call = functools.partial(
    pl.kernel,
    out_type=jax.ShapeDtypeStruct((MNPAD, H), jnp.float32),
    mesh=_mesh,
    scratch_types=[
        pltpu.VMEM((MNW,), jnp.int32),
        pltpu.VMEM((MNW,), jnp.int32),
        pltpu.VMEM((CHUNK, H), jnp.float32),
        pltpu.VMEM((CHUNK, H), jnp.float32),
        pltpu.SemaphoreType.DMA,
        pltpu.SemaphoreType.DMA,
    ],
)(_mn_body)


BLK = 1024


def _tc_b_body(degp_ref, x_ref, w_ref, hws_ref, dinv_ref):
    deg = degp_ref[0, :, :1] + degp_ref[1, :, :1]
    dinv = lax.rsqrt(deg + 1.0)
    hw = jnp.dot(x_ref[...], w_ref[...], preferred_element_type=jnp.float32)
    hws_ref[...] = hw * dinv
    dinv_ref[...] = dinv


def _tc_b(deg_parts, x_p, W1):
    return pl.pallas_call(
        _tc_b_body,
        grid=(NPAD // BLK,),
        in_specs=[
            pl.BlockSpec((2, BLK, DEGW), lambda i: (0, i, 0)),
            pl.BlockSpec((BLK, D), lambda i: (i, 0)),
            pl.BlockSpec((D, H), lambda i: (0, 0)),
        ],
        out_specs=[
            pl.BlockSpec((BLK, H), lambda i: (i, 0)),
            pl.BlockSpec((BLK, 1), lambda i: (i, 0)),
        ],
        out_shape=[
            jax.ShapeDtypeStruct((NPAD, H), jnp.float32),
            jax.ShapeDtypeStruct((NPAD, 1), jnp.float32),
        ],
    )(deg_parts, x_p, W1)


def _tc_layer_body2(p_ref, hws_ref, dinv_ref, b_ref, w_ref, out_ref, *,
                    scale_out):
    dinv = dinv_ref[...]
    h = jnp.maximum(
        dinv * (p_ref[0] + p_ref[1] + hws_ref[...]) + b_ref[...], 0.0)
    hw = jnp.dot(h, w_ref[...], preferred_element_type=jnp.float32)
    out_ref[...] = hw * dinv if scale_out else hw


def _tc_layer2(P, hws, dinv, b, W, scale_out):
    return pl.pallas_call(
        functools.partial(_tc_layer_body2, scale_out=scale_out),
        grid=(NPAD // BLK,),
        in_specs=[
            pl.BlockSpec((2, BLK, H), lambda i: (0, i, 0)),
            pl.BlockSpec((BLK, H), lambda i: (i, 0)),
            pl.BlockSpec((BLK, 1), lambda i: (i, 0)),
            pl.BlockSpec((1, H), lambda i: (0, 0)),
            pl.BlockSpec((H, H), lambda i: (0, 0)),
        ],
        out_specs=pl.BlockSpec((BLK, H), lambda i: (i, 0)),
        out_shape=jax.ShapeDtypeStruct((NPAD, H), jnp.float32),
    )(P, hws, dinv, b.reshape(1, H), W)


EBLK = 2048


def _tc_g_body(ee_ref, bp1_ref, wp2_ref, bp2_ref, out_ref):
    z = jnp.maximum(ee_ref[...] + bp1_ref[...], 0.0)
    out_ref[...] = jnp.dot(z, wp2_ref[...],
                           preferred_element_type=jnp.float32) + bp2_ref[...]


def _tc_g(ee, bp1, Wp2, bp2):
    return pl.pallas_call(
        _tc_g_body,
        grid=(MNPAD // EBLK,),
        in_specs=[
            pl.BlockSpec((EBLK, H), lambda i: (i, 0)),
            pl.BlockSpec((1, H), lambda i: (0, 0)),
            pl.BlockSpec((H, 1), lambda i: (0, 0)),
            pl.BlockSpec((1, 1), lambda i: (0, 0)),
        ],
        out_specs=pl.BlockSpec((EBLK, 1), lambda i: (i, 0)),
        out_shape=jax.ShapeDtypeStruct((MNPAD, 1), jnp.float32),
    )(ee, bp1.reshape(1, H), Wp2, bp2.reshape(1, 1))


def kernel(x, edge_index, MN_edge_index, W1, b1, W2, b2, Wp1, bp1, Wp2, bp2):
    src = jnp.concatenate(
        [edge_index[0], jnp.zeros((EPAD - E,), jnp.int32)])
    dst = jnp.concatenate(
        [edge_index[1], jnp.full((EPAD - E,), N_NODES, jnp.int32)])
    dst3 = dst.reshape(NW, NCH, CHUNK)
    dst4 = dst.reshape(NW, SNCH, SCHUNK)
    a_idx = jnp.concatenate(
        [MN_edge_index[0], jnp.full((MNPAD - E_MN,), N_NODES, jnp.int32)])
    b_idx = jnp.concatenate(
        [MN_edge_index[1], jnp.full((MNPAD - E_MN,), N_NODES, jnp.int32)])
    x_p = jnp.pad(x, ((0, NPAD - N_NODES), (0, 0)))

    deg_parts = _deg_call(dst3)
    hws1, dinv = _tc_b(deg_parts, x_p, W1)
    P1 = _segsum_call(hws1, src, dst4)
    hws2 = _tc_layer2(P1, hws1, dinv, b1, W2, True)
    P2 = _segsum_call(hws2, src, dst4)
    hp = _tc_layer2(P2, hws2, dinv, b2, Wp1, False)
    ee = _mn_call(hp, a_idx, b_idx)
    pred = _tc_g(ee, bp1, Wp2, bp2)
    return pred[:E_MN, 0]


# speedup vs baseline: 8.8085x; 1.0251x over previous
import functools

import jax
import jax.numpy as jnp
from jax import lax
from jax.experimental import pallas as pl
from jax.experimental.pallas import tpu as pltpu
from jax.experimental.pallas import tpu_sc as plsc

N_NODES = 10000
D = 128
H = 128
E = 320000
E_MN = 100000

NPAD = 10240
NW = 32
CHUNK = 128
NCH = (E // NW) // CHUNK + 1
EW = NCH * CHUNK
EPAD = NW * EW
SCHUNK = 64
SNCH = EW // SCHUNK
MNCH = (E_MN // NW) // CHUNK + 1
MNW = MNCH * CHUNK
MNPAD = NW * MNW

_mesh = plsc.VectorSubcoreMesh(core_axis_name="c", subcore_axis_name="s",
                               num_cores=2, num_subcores=16)
ROWS_PER_TILE = NPAD // 16


DEGW = 16


def _deg_body(dst_hbm, out_hbm, dst_v, ones_v, zero_v, accum_sh, isem):
    c = lax.axis_index("c")
    s = lax.axis_index("s")
    icp = pltpu.async_copy(dst_hbm.at[c * 16 + s], dst_v, isem)

    ones16 = jnp.ones((16,), jnp.float32)
    zeros16 = jnp.zeros((16,), jnp.float32)

    def fill(i, _):
        ones_v[i, pl.ds(0, 16)] = ones16
        zero_v[i % zero_v.shape[0], pl.ds(0, 16)] = zeros16
        return ()

    lax.fori_loop(0, CHUNK, fill, (), unroll=8)

    def zcopy(i, _):
        pltpu.sync_copy(
            zero_v, accum_sh.at[pl.ds(s * ROWS_PER_TILE + i * zero_v.shape[0],
                                      zero_v.shape[0])])
        return ()

    lax.fori_loop(0, ROWS_PER_TILE // zero_v.shape[0], zcopy, ())
    icp.wait()
    plsc.subcore_barrier()

    def body(g, _):
        pltpu.sync_copy(ones_v, accum_sh.at[dst_v.at[g]], add=True)
        return ()

    lax.fori_loop(0, NCH, body, ())
    plsc.subcore_barrier()
    pltpu.sync_copy(accum_sh.at[pl.ds(s * ROWS_PER_TILE, ROWS_PER_TILE)],
                    out_hbm.at[c, pl.ds(s * ROWS_PER_TILE, ROWS_PER_TILE)])


_deg_call = functools.partial(
    pl.kernel,
    out_type=jax.ShapeDtypeStruct((2, NPAD, DEGW), jnp.float32),
    mesh=_mesh,
    scratch_types=[
        pltpu.VMEM((NCH, CHUNK), jnp.int32),
        pltpu.VMEM((CHUNK, DEGW), jnp.float32),
        pltpu.VMEM((64, DEGW), jnp.float32),
        pltpu.VMEM_SHARED((NPAD, DEGW), jnp.float32),
        pltpu.SemaphoreType.DMA,
    ],
)(_deg_body)


def _segsum_body(table_hbm, src_hbm, dst_hbm, out_hbm,
                 src_v, dstb_v, rows_v, zero_v, accum_sh, gsem, isem):
    c = lax.axis_index("c")
    s = lax.axis_index("s")
    wid = c * 16 + s

    icp = pltpu.async_copy(src_hbm.at[pl.ds(wid * EW, EW)], src_v, isem)
    dcp = pltpu.async_copy(dst_hbm.at[wid], dstb_v, isem)

    zeros16 = jnp.zeros((16,), jnp.float32)

    def zbody(i, _):
        zero_v[i // 8, pl.ds((i % 8) * 16, 16)] = zeros16
        return ()

    lax.fori_loop(0, zero_v.shape[0] * 8, zbody, (), unroll=8)

    def zcopy(i, _):
        pltpu.sync_copy(
            zero_v, accum_sh.at[pl.ds(s * ROWS_PER_TILE + i * zero_v.shape[0],
                                      zero_v.shape[0])])
        return ()

    lax.fori_loop(0, ROWS_PER_TILE // zero_v.shape[0], zcopy, ())
    icp.wait()
    dcp.wait()
    plsc.subcore_barrier()

    pltpu.async_copy(table_hbm.at[src_v.at[pl.ds(0, SCHUNK)]], rows_v.at[0],
                     gsem)

    def body(g, _):
        buf = lax.rem(g, 2)
        pltpu.make_async_copy(table_hbm.at[src_v.at[pl.ds(0, SCHUNK)]],
                              rows_v.at[buf], gsem).wait()

        @pl.when(g + 1 < SNCH)
        def _():
            pltpu.async_copy(
                table_hbm.at[src_v.at[pl.ds((g + 1) * SCHUNK, SCHUNK)]],
                rows_v.at[lax.rem(g + 1, 2)], gsem)

        pltpu.sync_copy(rows_v.at[buf], accum_sh.at[dstb_v.at[g]], add=True)
        return ()

    lax.fori_loop(0, SNCH, body, ())
    plsc.subcore_barrier()
    pltpu.sync_copy(accum_sh.at[pl.ds(s * ROWS_PER_TILE, ROWS_PER_TILE)],
                    out_hbm.at[c, pl.ds(s * ROWS_PER_TILE, ROWS_PER_TILE)])


_segsum_call = functools.partial(
    pl.kernel,
    out_type=jax.ShapeDtypeStruct((2, NPAD, H), jnp.float32),
    mesh=_mesh,
    scratch_types=[
        pltpu.VMEM((EW,), jnp.int32),
        pltpu.VMEM((SNCH, SCHUNK), jnp.int32),
        pltpu.VMEM((2, SCHUNK, H), jnp.float32),
        pltpu.VMEM((16, H), jnp.float32),
        pltpu.VMEM_SHARED((NPAD, H), jnp.float32),
        pltpu.SemaphoreType.DMA,
        pltpu.SemaphoreType.DMA,
    ],
)(_segsum_body)


def _mn_body(table_hbm, a_hbm, b_hbm, out_hbm,
             a_v, b_v, rows_a, rows_b, gsem, wsem, isem):
    c = lax.axis_index("c")
    s = lax.axis_index("s")
    wid = c * 16 + s
    pltpu.async_copy(a_hbm.at[pl.ds(wid * MNW, MNW)], a_v, isem)
    pltpu.async_copy(b_hbm.at[pl.ds(wid * MNW, MNW)], b_v, isem)
    pltpu.make_async_copy(a_hbm.at[pl.ds(wid * MNW, MNW)], a_v, isem).wait()
    pltpu.make_async_copy(b_hbm.at[pl.ds(wid * MNW, MNW)], b_v, isem).wait()

    def gstart(g, buf):
        pltpu.async_copy(table_hbm.at[a_v.at[pl.ds(g * CHUNK, CHUNK)]],
                         rows_a.at[buf], gsem)
        pltpu.async_copy(table_hbm.at[b_v.at[pl.ds(g * CHUNK, CHUNK)]],
                         rows_b.at[buf], gsem)

    gstart(0, 0)

    def body(g, _):
        buf = lax.rem(g, 2)

        @pl.when(g >= 1)
        def _():
            pltpu.make_async_copy(
                rows_a.at[lax.rem(g + 1, 2)],
                out_hbm.at[pl.ds(wid * MNW, CHUNK)], wsem).wait()

        pltpu.make_async_copy(table_hbm.at[a_v.at[pl.ds(0, CHUNK)]],
                              rows_a.at[buf], gsem).wait()
        pltpu.make_async_copy(table_hbm.at[b_v.at[pl.ds(0, CHUNK)]],
                              rows_b.at[buf], gsem).wait()

        @pl.when(g + 1 < MNCH)
        def _():
            gstart(g + 1, lax.rem(g + 1, 2))

        def add_body(r, _):
            for k in range(H // 16):
                sl = pl.ds(k * 16, 16)
                rows_a[buf, r, sl] = rows_a[buf, r, sl] + rows_b[buf, r, sl]
            return ()

        lax.fori_loop(0, CHUNK, add_body, (), unroll=4)
        pltpu.async_copy(rows_a.at[buf],
                         out_hbm.at[pl.ds(wid * MNW + g * CHUNK, CHUNK)], wsem)
        return ()

    lax.fori_loop(0, MNCH, body, ())
    pltpu.make_async_copy(rows_a.at[lax.rem(MNCH - 1, 2)],
                          out_hbm.at[pl.ds(wid * MNW, CHUNK)], wsem).wait()


_mn_call = functools.partial(
    pl.kernel,
    out_type=jax.ShapeDtypeStruct((MNPAD, H), jnp.float32),
    mesh=_mesh,
    scratch_types=[
        pltpu.VMEM((MNW,), jnp.int32),
        pltpu.VMEM((MNW,), jnp.int32),
        pltpu.VMEM((2, CHUNK, H), jnp.float32),
        pltpu.VMEM((2, CHUNK, H), jnp.float32),
        pltpu.SemaphoreType.DMA,
        pltpu.SemaphoreType.DMA,
        pltpu.SemaphoreType.DMA,
    ],
)(_mn_body)


BLK = 1024


def _tc_b_body(degp_ref, x_ref, w_ref, hws_ref, dinv_ref):
    deg = degp_ref[0, :, :1] + degp_ref[1, :, :1]
    dinv = lax.rsqrt(deg + 1.0)
    hw = jnp.dot(x_ref[...], w_ref[...], preferred_element_type=jnp.float32)
    hws_ref[...] = hw * dinv
    dinv_ref[...] = dinv


def _tc_b(deg_parts, x_p, W1):
    return pl.pallas_call(
        _tc_b_body,
        grid=(NPAD // BLK,),
        in_specs=[
            pl.BlockSpec((2, BLK, DEGW), lambda i: (0, i, 0)),
            pl.BlockSpec((BLK, D), lambda i: (i, 0)),
            pl.BlockSpec((D, H), lambda i: (0, 0)),
        ],
        out_specs=[
            pl.BlockSpec((BLK, H), lambda i: (i, 0)),
            pl.BlockSpec((BLK, 1), lambda i: (i, 0)),
        ],
        out_shape=[
            jax.ShapeDtypeStruct((NPAD, H), jnp.float32),
            jax.ShapeDtypeStruct((NPAD, 1), jnp.float32),
        ],
    )(deg_parts, x_p, W1)


def _tc_layer_body2(p_ref, hws_ref, dinv_ref, b_ref, w_ref, out_ref, *,
                    scale_out):
    dinv = dinv_ref[...]
    h = jnp.maximum(
        dinv * (p_ref[0] + p_ref[1] + hws_ref[...]) + b_ref[...], 0.0)
    hw = jnp.dot(h, w_ref[...], preferred_element_type=jnp.float32)
    out_ref[...] = hw * dinv if scale_out else hw


def _tc_layer2(P, hws, dinv, b, W, scale_out):
    return pl.pallas_call(
        functools.partial(_tc_layer_body2, scale_out=scale_out),
        grid=(NPAD // BLK,),
        in_specs=[
            pl.BlockSpec((2, BLK, H), lambda i: (0, i, 0)),
            pl.BlockSpec((BLK, H), lambda i: (i, 0)),
            pl.BlockSpec((BLK, 1), lambda i: (i, 0)),
            pl.BlockSpec((1, H), lambda i: (0, 0)),
            pl.BlockSpec((H, H), lambda i: (0, 0)),
        ],
        out_specs=pl.BlockSpec((BLK, H), lambda i: (i, 0)),
        out_shape=jax.ShapeDtypeStruct((NPAD, H), jnp.float32),
    )(P, hws, dinv, b.reshape(1, H), W)


EBLK = 2048


def _tc_g_body(ee_ref, bp1_ref, wp2_ref, bp2_ref, out_ref):
    z = jnp.maximum(ee_ref[...] + bp1_ref[...], 0.0)
    out_ref[...] = jnp.dot(z, wp2_ref[...],
                           preferred_element_type=jnp.float32) + bp2_ref[...]


def _tc_g(ee, bp1, Wp2, bp2):
    return pl.pallas_call(
        _tc_g_body,
        grid=(MNPAD // EBLK,),
        in_specs=[
            pl.BlockSpec((EBLK, H), lambda i: (i, 0)),
            pl.BlockSpec((1, H), lambda i: (0, 0)),
            pl.BlockSpec((H, 1), lambda i: (0, 0)),
            pl.BlockSpec((1, 1), lambda i: (0, 0)),
        ],
        out_specs=pl.BlockSpec((EBLK, 1), lambda i: (i, 0)),
        out_shape=jax.ShapeDtypeStruct((MNPAD, 1), jnp.float32),
    )(ee, bp1.reshape(1, H), Wp2, bp2.reshape(1, 1))


def kernel(x, edge_index, MN_edge_index, W1, b1, W2, b2, Wp1, bp1, Wp2, bp2):
    src = jnp.concatenate(
        [edge_index[0], jnp.zeros((EPAD - E,), jnp.int32)])
    dst = jnp.concatenate(
        [edge_index[1], jnp.full((EPAD - E,), N_NODES, jnp.int32)])
    dst3 = dst.reshape(NW, NCH, CHUNK)
    a_idx = jnp.concatenate(
        [MN_edge_index[0], jnp.full((MNPAD - E_MN,), N_NODES, jnp.int32)])
    b_idx = jnp.concatenate(
        [MN_edge_index[1], jnp.full((MNPAD - E_MN,), N_NODES, jnp.int32)])
    x_p = jnp.pad(x, ((0, NPAD - N_NODES), (0, 0)))

    deg_parts = _deg_call(dst3)
    hws1, dinv = _tc_b(deg_parts, x_p, W1)
    dst4 = dst.reshape(NW, SNCH, SCHUNK)
    P1 = _segsum_call(hws1, src, dst4)
    hws2 = _tc_layer2(P1, hws1, dinv, b1, W2, True)
    P2 = _segsum_call(hws2, src, dst4)
    hp = _tc_layer2(P2, hws2, dinv, b2, Wp1, False)
    ee = _mn_call(hp, a_idx, b_idx)
    pred = _tc_g(ee, bp1, Wp2, bp2)
    return pred[:E_MN, 0]
